# baseline (speedup 1.0000x reference)
import jax
import jax.numpy as jnp
from jax.experimental import pallas as pl
from jax.experimental.pallas import tpu as pltpu

_NUM_NODES = 1000000
_DIM = 64
_NCHUNKS = 8
_ROWS_PER_CHUNK = _NUM_NODES // _NCHUNKS


def _copy_body(x_hbm, o_hbm, sems):
    copies = []
    for i in range(_NCHUNKS):
        sl = pl.ds(i * _ROWS_PER_CHUNK, _ROWS_PER_CHUNK)
        copies.append(
            pltpu.make_async_copy(x_hbm.at[sl, :], o_hbm.at[sl, :], sems.at[i])
        )
    for c in copies:
        c.start()
    for c in copies:
        c.wait()


def kernel(emb_weight):
    return pl.pallas_call(
        _copy_body,
        out_shape=jax.ShapeDtypeStruct((_NUM_NODES, _DIM), jnp.float32),
        in_specs=[pl.BlockSpec(memory_space=pltpu.MemorySpace.HBM)],
        out_specs=pl.BlockSpec(memory_space=pltpu.MemorySpace.HBM),
        scratch_shapes=[pltpu.SemaphoreType.DMA((_NCHUNKS,))],
    )(emb_weight)


# speedup vs baseline: 11.8595x; 11.8595x over previous
import jax
import jax.numpy as jnp
from jax.experimental import pallas as pl

_NUM_NODES = 1000000
_DIM = 64
_TOTAL = _NUM_NODES * _DIM
_BLOCK = 2 * 1024 * 1024


def _copy_block(x_ref, o_ref):
    o_ref[...] = x_ref[...]


def kernel(emb_weight):
    flat = emb_weight.reshape(_TOTAL)
    out = pl.pallas_call(
        _copy_block,
        out_shape=jax.ShapeDtypeStruct((_TOTAL,), jnp.float32),
        grid=(_TOTAL // _BLOCK,),
        in_specs=[pl.BlockSpec((_BLOCK,), lambda i: (i,))],
        out_specs=pl.BlockSpec((_BLOCK,), lambda i: (i,)),
    )(flat)
    return out.reshape(_NUM_NODES, _DIM)
